# baseline (speedup 1.0000x reference)
import functools

import jax
import jax.numpy as jnp
from jax import lax
from jax.experimental import pallas as pl
from jax.experimental.pallas import tpu as pltpu
from jax.experimental.pallas import tpu_sc as plsc

N_NODES = 10000
N_EDGES = 320000
NC = 2
NS = 16
NW = NC * NS
EPW = N_EDGES // NW
K = 128
KH = K // 2
NWIN = EPW // K
KT = EPW - NWIN * K


def _mm_body(a_ref, b_ref, o_ref):
    o_ref[...] = jnp.dot(a_ref[...], b_ref[...],
                         preferred_element_type=jnp.float32,
                         precision=lax.Precision.HIGHEST)


def _matmul(a, b):
    return pl.pallas_call(
        _mm_body,
        out_shape=jax.ShapeDtypeStruct((a.shape[0], b.shape[1]), jnp.float32),
    )(a, b)


def _mm_combine_body(p_ref, w_ref, o_ref):
    h = p_ref[0] + p_ref[1]
    o_ref[...] = jnp.dot(h, w_ref[...],
                         preferred_element_type=jnp.float32,
                         precision=lax.Precision.HIGHEST)


def _matmul_combine(p, w):
    n, d = p.shape[1], p.shape[2]
    blk = n // 10
    return pl.pallas_call(
        _mm_combine_body,
        grid=(10,),
        in_specs=[pl.BlockSpec((2, blk, d), lambda i: (0, i, 0)),
                  pl.BlockSpec((d, w.shape[1]), lambda i: (0, 0))],
        out_specs=pl.BlockSpec((blk, w.shape[1]), lambda i: (i, 0)),
        out_shape=jax.ShapeDtypeStruct((n, w.shape[1]), jnp.float32),
    )(p, w)


def _add_body(p_ref, o_ref):
    o_ref[...] = p_ref[0] + p_ref[1]


def _add_partials(p):
    n, d = p.shape[1], p.shape[2]
    blk = n // 10
    return pl.pallas_call(
        _add_body,
        grid=(10,),
        in_specs=[pl.BlockSpec((2, blk, d), lambda i: (0, i, 0))],
        out_specs=pl.BlockSpec((blk, d), lambda i: (i, 0)),
        out_shape=jax.ShapeDtypeStruct((n, d), jnp.float32),
    )(p)


def _sc_aggregate(h, src2, dst2):
    n, d = h.shape
    rows_per_tile = (n // NS) // 8 * 8 + 8
    mesh = plsc.VectorSubcoreMesh(core_axis_name="c", subcore_axis_name="s")

    @functools.partial(
        pl.kernel,
        out_type=jax.ShapeDtypeStruct((NC, n, d), jnp.float32),
        mesh=mesh,
        scratch_types=(
            [pltpu.VMEM_SHARED((n, d), jnp.float32)]
            + [pltpu.VMEM((K,), jnp.int32)] * 4
            + [pltpu.VMEM((KT,), jnp.int32)] * 2
            + [pltpu.VMEM((K, d), jnp.float32)] * 2
            + [pltpu.SemaphoreType.DMA] * 10
        ),
    )
    def agg(h_hbm, src_hbm, dst_hbm, z_hbm, out_hbm, acc,
            sv0, sv1, dv0, dv1, svt, dvt, rb0, rb1,
            f0, f1, e0, e1, g0, g1, h0, h1, t0, t1):
        srcv = (sv0, sv1)
        dstv = (dv0, dv1)
        rows = (rb0, rb1)
        fs = (f0, f1)
        es = (e0, e1)
        gs = (g0, g1)
        hs = (h0, h1)
        ss = (t0, t1)
        c = lax.axis_index("c")
        s = lax.axis_index("s")
        wid = c * NS + s

        def src_slice(w):
            return src_hbm.at[wid, pl.ds(pl.multiple_of(w * K, K), K)]

        def dst_slice(w):
            return dst_hbm.at[wid, pl.ds(pl.multiple_of(w * K, K), K)]

        def gather_start(j):
            pltpu.async_copy(h_hbm.at[srcv[j].at[pl.ds(0, KH)]],
                             rows[j].at[pl.ds(0, KH)], gs[j])
            pltpu.async_copy(h_hbm.at[srcv[j].at[pl.ds(KH, KH)]],
                             rows[j].at[pl.ds(KH, KH)], hs[j])

        def gather_wait(j):
            pltpu.make_async_copy(h_hbm.at[srcv[j].at[pl.ds(0, KH)]],
                                  rows[j].at[pl.ds(0, KH)], gs[j]).wait()
            pltpu.make_async_copy(h_hbm.at[srcv[j].at[pl.ds(KH, KH)]],
                                  rows[j].at[pl.ds(KH, KH)], hs[j]).wait()

        def scatter_wait(j):
            pltpu.make_async_copy(rows[j], acc.at[dstv[j]], ss[j]).wait()

        def visit(w, j, first=False, last=False):
            j2 = 1 - j
            gather_wait(j)
            if not last:
                pltpu.async_copy(src_slice(w + 2), srcv[j], fs[j])
            pltpu.make_async_copy(dst_slice(w), dstv[j], es[j]).wait()
            pltpu.async_copy(rows[j], acc.at[dstv[j]], ss[j], add=True)
            if not last:
                pltpu.make_async_copy(src_slice(w + 1), srcv[j2], fs[j2]).wait()
                if not first:
                    scatter_wait(j2)
                pltpu.async_copy(dst_slice(w + 1), dstv[j2], es[j2])
                gather_start(j2)

        pltpu.sync_copy(src_slice(0), srcv[0])
        pltpu.async_copy(dst_slice(0), dstv[0], es[0])
        gather_start(0)
        pltpu.async_copy(src_slice(1), srcv[1], fs[1])
        r0 = pl.multiple_of(s * (n // NS) // 8 * 8, 8)
        pltpu.sync_copy(z_hbm.at[pl.ds(r0, rows_per_tile)],
                        acc.at[pl.ds(r0, rows_per_tile)])
        plsc.subcore_barrier()

        visit(0, 0, first=True)

        @pl.loop(1, NWIN - 3, step=2)
        def _(w):
            visit(w, 1)
            visit(w + 1, 0)

        visit(NWIN - 3, 1)
        visit(NWIN - 2, 0, last=True)
        pltpu.make_async_copy(src_slice(NWIN - 1), srcv[1], fs[1]).wait()
        scatter_wait(1)
        pltpu.async_copy(dst_slice(NWIN - 1), dstv[1], es[1])
        gather_start(1)
        gather_wait(1)
        pltpu.make_async_copy(dst_slice(NWIN - 1), dstv[1], es[1]).wait()
        pltpu.async_copy(rows[1], acc.at[dstv[1]], ss[1], add=True)
        scatter_wait(0)
        scatter_wait(1)

        toff = pl.multiple_of(NWIN * K, 8)
        pltpu.sync_copy(src_hbm.at[wid, pl.ds(toff, KT)], svt)
        pltpu.sync_copy(dst_hbm.at[wid, pl.ds(toff, KT)], dvt)
        pltpu.sync_copy(h_hbm.at[svt], rows[0].at[pl.ds(0, KT)])
        pltpu.sync_copy(rows[0].at[pl.ds(0, KT)], acc.at[dvt], add=True)

        plsc.subcore_barrier()
        pltpu.sync_copy(acc.at[pl.ds(r0, rows_per_tile)],
                        out_hbm.at[c, pl.ds(r0, rows_per_tile)])

    zeros = jnp.zeros((n, d), jnp.float32)
    return agg(h, src2, dst2, zeros)


def kernel(x, edge_index, W1, W2):
    ei = edge_index.astype(jnp.int32)
    src2 = ei[0].reshape(NW, EPW)
    dst2 = ei[1].reshape(NW, EPW)

    w12 = _matmul(W1, W2)
    p1 = _sc_aggregate(x, src2, dst2)
    t1 = _add_partials(p1)
    p2 = _sc_aggregate(t1, src2, dst2)
    return _matmul_combine(p2, w12)


# speedup vs baseline: 1.0242x; 1.0242x over previous
import functools

import jax
import jax.numpy as jnp
from jax import lax
from jax.experimental import pallas as pl
from jax.experimental.pallas import tpu as pltpu
from jax.experimental.pallas import tpu_sc as plsc

N_NODES = 10000
N_EDGES = 320000
NC = 2
NS = 16
NW = NC * NS
EPW = N_EDGES // NW
K = 128
KH = K // 2
NWIN = EPW // K
KT = EPW - NWIN * K


def _mm_body(a_ref, b_ref, o_ref):
    o_ref[...] = jnp.dot(a_ref[...], b_ref[...],
                         preferred_element_type=jnp.float32,
                         precision=lax.Precision.HIGHEST)


def _matmul(a, b):
    return pl.pallas_call(
        _mm_body,
        out_shape=jax.ShapeDtypeStruct((a.shape[0], b.shape[1]), jnp.float32),
    )(a, b)


def _mm_combine_body(p_ref, w_ref, o_ref):
    h = p_ref[0] + p_ref[1]
    o_ref[...] = jnp.dot(h, w_ref[...],
                         preferred_element_type=jnp.float32,
                         precision=lax.Precision.HIGHEST)


def _matmul_combine(p, w):
    return pl.pallas_call(
        _mm_combine_body,
        out_shape=jax.ShapeDtypeStruct((p.shape[1], w.shape[1]), jnp.float32),
    )(p, w)


def _add_body(p_ref, o_ref):
    o_ref[...] = p_ref[0] + p_ref[1]


def _add_partials(p):
    return pl.pallas_call(
        _add_body,
        out_shape=jax.ShapeDtypeStruct(p.shape[1:], jnp.float32),
    )(p)


def _sc_aggregate(h, src2, dst2):
    n, d = h.shape
    rows_per_tile = (n // NS) // 8 * 8 + 8
    mesh = plsc.VectorSubcoreMesh(core_axis_name="c", subcore_axis_name="s")

    @functools.partial(
        pl.kernel,
        out_type=jax.ShapeDtypeStruct((NC, n, d), jnp.float32),
        mesh=mesh,
        scratch_types=(
            [pltpu.VMEM_SHARED((n, d), jnp.float32)]
            + [pltpu.VMEM((K,), jnp.int32)] * 4
            + [pltpu.VMEM((KT,), jnp.int32)] * 2
            + [pltpu.VMEM((K, d), jnp.float32)] * 2
            + [pltpu.SemaphoreType.DMA] * 10
        ),
    )
    def agg(h_hbm, src_hbm, dst_hbm, z_hbm, out_hbm, acc,
            sv0, sv1, dv0, dv1, svt, dvt, rb0, rb1,
            f0, f1, e0, e1, g0, g1, h0, h1, t0, t1):
        srcv = (sv0, sv1)
        dstv = (dv0, dv1)
        rows = (rb0, rb1)
        fs = (f0, f1)
        es = (e0, e1)
        gs = (g0, g1)
        hs = (h0, h1)
        ss = (t0, t1)
        c = lax.axis_index("c")
        s = lax.axis_index("s")
        wid = c * NS + s

        def src_slice(w):
            return src_hbm.at[wid, pl.ds(pl.multiple_of(w * K, K), K)]

        def dst_slice(w):
            return dst_hbm.at[wid, pl.ds(pl.multiple_of(w * K, K), K)]

        def gather_start(j):
            pltpu.async_copy(h_hbm.at[srcv[j].at[pl.ds(0, KH)]],
                             rows[j].at[pl.ds(0, KH)], gs[j])
            pltpu.async_copy(h_hbm.at[srcv[j].at[pl.ds(KH, KH)]],
                             rows[j].at[pl.ds(KH, KH)], hs[j])

        def gather_wait(j):
            pltpu.make_async_copy(h_hbm.at[srcv[j].at[pl.ds(0, KH)]],
                                  rows[j].at[pl.ds(0, KH)], gs[j]).wait()
            pltpu.make_async_copy(h_hbm.at[srcv[j].at[pl.ds(KH, KH)]],
                                  rows[j].at[pl.ds(KH, KH)], hs[j]).wait()

        def scatter_wait(j):
            pltpu.make_async_copy(rows[j], acc.at[dstv[j]], ss[j]).wait()

        def visit(w, j, first=False, last=False):
            j2 = 1 - j
            gather_wait(j)
            if not last:
                pltpu.async_copy(src_slice(w + 2), srcv[j], fs[j])
            pltpu.make_async_copy(dst_slice(w), dstv[j], es[j]).wait()
            pltpu.async_copy(rows[j], acc.at[dstv[j]], ss[j], add=True)
            if not last:
                pltpu.make_async_copy(src_slice(w + 1), srcv[j2], fs[j2]).wait()
                if not first:
                    scatter_wait(j2)
                pltpu.async_copy(dst_slice(w + 1), dstv[j2], es[j2])
                gather_start(j2)

        pltpu.sync_copy(src_slice(0), srcv[0])
        pltpu.async_copy(dst_slice(0), dstv[0], es[0])
        gather_start(0)
        pltpu.async_copy(src_slice(1), srcv[1], fs[1])
        r0 = pl.multiple_of(s * (n // NS) // 8 * 8, 8)
        pltpu.sync_copy(z_hbm.at[pl.ds(r0, rows_per_tile)],
                        acc.at[pl.ds(r0, rows_per_tile)])
        plsc.subcore_barrier()

        visit(0, 0, first=True)

        @pl.loop(1, NWIN - 3, step=2)
        def _(w):
            visit(w, 1)
            visit(w + 1, 0)

        visit(NWIN - 3, 1)
        visit(NWIN - 2, 0, last=True)
        pltpu.make_async_copy(src_slice(NWIN - 1), srcv[1], fs[1]).wait()
        scatter_wait(1)
        pltpu.async_copy(dst_slice(NWIN - 1), dstv[1], es[1])
        gather_start(1)
        gather_wait(1)
        pltpu.make_async_copy(dst_slice(NWIN - 1), dstv[1], es[1]).wait()
        pltpu.async_copy(rows[1], acc.at[dstv[1]], ss[1], add=True)
        scatter_wait(0)
        scatter_wait(1)

        toff = pl.multiple_of(NWIN * K, 8)
        pltpu.sync_copy(src_hbm.at[wid, pl.ds(toff, KT)], svt)
        pltpu.sync_copy(dst_hbm.at[wid, pl.ds(toff, KT)], dvt)
        pltpu.sync_copy(h_hbm.at[svt], rows[0].at[pl.ds(0, KT)])
        pltpu.sync_copy(rows[0].at[pl.ds(0, KT)], acc.at[dvt], add=True)

        plsc.subcore_barrier()
        pltpu.sync_copy(acc.at[pl.ds(r0, rows_per_tile)],
                        out_hbm.at[c, pl.ds(r0, rows_per_tile)])

    zeros = jnp.zeros((n, d), jnp.float32)
    return agg(h, src2, dst2, zeros)


def kernel(x, edge_index, W1, W2):
    ei = edge_index.astype(jnp.int32)
    src2 = ei[0].reshape(NW, EPW)
    dst2 = ei[1].reshape(NW, EPW)

    w12 = _matmul(W1, W2)
    p1 = _sc_aggregate(x, src2, dst2)
    t1 = _add_partials(p1)
    p2 = _sc_aggregate(t1, src2, dst2)
    return _matmul_combine(p2, w12)
